# Initial kernel scaffold; baseline (speedup 1.0000x reference)
import jax
import jax.numpy as jnp
from jax.experimental import pallas as pl


def kernel(feat, edge_index, weight1, weight2, loop_weight, bias):
    raise NotImplementedError("write your pallas kernel here")


# speedup vs baseline: 7.2183x; 7.2183x over previous
import functools

import jax
import jax.numpy as jnp
from jax import lax
from jax.experimental import pallas as pl
from jax.experimental.pallas import tpu as pltpu
from jax.experimental.pallas import tpu_sc as plsc

NC = 2
NS = 16
NW = NC * NS
LANES = 16

CHUNK = 80


def _ab_body(feat_ref, w1a_ref, w1b_ref, a_ref, b_ref):
    f = feat_ref[...]
    a_ref[...] = jnp.dot(f, w1a_ref[...], preferred_element_type=jnp.float32)
    b_ref[...] = jnp.dot(f, w1b_ref[...], preferred_element_type=jnp.float32)


def _final_body(s_ref, feat_ref, w2_ref, lw_ref, bias_ref, o_ref):
    s = s_ref[0] + s_ref[1]
    o_ref[...] = (
        jnp.dot(s, w2_ref[...], preferred_element_type=jnp.float32)
        + jnp.dot(feat_ref[...], lw_ref[...], preferred_element_type=jnp.float32)
        + bias_ref[...]
    )


def _make_sc_edge_kernel(n_nodes, d, n_chunks_total):
    chunks_per_worker = n_chunks_total // NW
    rows_per_tile = (n_nodes // NS) & ~7
    rows_rem = n_nodes - rows_per_tile * NS
    pieces = []
    off = 0
    while off < rows_per_tile:
        sz = min(CHUNK, rows_per_tile - off)
        pieces.append((off, sz))
        off += sz
    idx_rows = min(64, chunks_per_worker)
    phases = []
    off = 0
    while off < chunks_per_worker:
        sz = min(idx_rows, chunks_per_worker - off)
        phases.append((off, sz))
        off += sz
    mesh = plsc.VectorSubcoreMesh(core_axis_name="c", subcore_axis_name="s")

    @functools.partial(
        pl.kernel,
        out_type=jax.ShapeDtypeStruct((NC, n_nodes, d), jnp.float32),
        mesh=mesh,
        scratch_types=[
            pltpu.VMEM((idx_rows, CHUNK), jnp.int32),
            pltpu.VMEM((idx_rows, CHUNK), jnp.int32),
            pltpu.VMEM((CHUNK, d), jnp.float32),
            pltpu.VMEM((CHUNK, d), jnp.float32),
            pltpu.VMEM_SHARED((n_nodes, d), jnp.float32),
            pltpu.SemaphoreType.DMA,
            pltpu.SemaphoreType.DMA,
        ],
    )
    def sc_edge(a_hbm, b_hbm, src_hbm, dst_hbm, out_hbm,
                idx_s, idx_d, rows_a, rows_b, acc, sem_a, sem_b):
        cid = lax.axis_index("c")
        sid = lax.axis_index("s")
        wid = sid * NC + cid
        tile_base = sid * rows_per_tile

        zero = jnp.zeros((LANES,), jnp.float32)

        def zrow(r, carry):
            for cix in range(d // LANES):
                rows_a[r, pl.ds(cix * LANES, LANES)] = zero
            return carry

        lax.fori_loop(0, CHUNK, zrow, 0)
        for poff, psz in pieces:
            pltpu.sync_copy(rows_a.at[pl.ds(0, psz)],
                            acc.at[pl.ds(tile_base + poff, psz)])
        if rows_rem:
            @pl.when(sid == NS - 1)
            def _zero_rem():
                pltpu.sync_copy(rows_a.at[pl.ds(0, rows_rem)],
                                acc.at[pl.ds(NS * rows_per_tile, rows_rem)])
        plsc.subcore_barrier()

        def step(i, carry):
            ca = pltpu.async_copy(a_hbm.at[idx_s.at[i]], rows_a, sem_a)
            cb = pltpu.async_copy(b_hbm.at[idx_d.at[i]], rows_b, sem_b)
            ca.wait()
            cb.wait()

            def vrow(r, c2):
                for cix in range(d // LANES):
                    sl = pl.ds(cix * LANES, LANES)
                    rows_b[r, sl] = jnp.maximum(rows_a[r, sl] + rows_b[r, sl], 0.0)
                return c2

            lax.fori_loop(0, CHUNK, vrow, 0)
            pltpu.sync_copy(rows_b, acc.at[idx_d.at[i]], add=True)
            return carry

        for phoff, phsz in phases:
            pltpu.sync_copy(src_hbm.at[wid].at[pl.ds(phoff, phsz)],
                            idx_s.at[pl.ds(0, phsz)])
            pltpu.sync_copy(dst_hbm.at[wid].at[pl.ds(phoff, phsz)],
                            idx_d.at[pl.ds(0, phsz)])
            lax.fori_loop(0, phsz, step, 0)

        plsc.subcore_barrier()
        for poff, psz in pieces:
            pltpu.sync_copy(acc.at[pl.ds(tile_base + poff, psz)],
                            rows_a.at[pl.ds(0, psz)])
            pltpu.sync_copy(rows_a.at[pl.ds(0, psz)],
                            out_hbm.at[cid].at[pl.ds(tile_base + poff, psz)])
        if rows_rem:
            @pl.when(sid == NS - 1)
            def _copy_rem():
                pltpu.sync_copy(acc.at[pl.ds(NS * rows_per_tile, rows_rem)],
                                rows_a.at[pl.ds(0, rows_rem)])
                pltpu.sync_copy(rows_a.at[pl.ds(0, rows_rem)],
                                out_hbm.at[cid].at[pl.ds(NS * rows_per_tile, rows_rem)])

    return sc_edge


def kernel(feat, edge_index, weight1, weight2, loop_weight, bias):
    n, d = feat.shape
    e = edge_index.shape[1]
    w1a = weight1[:d]
    w1b = weight1[d:]

    row_block = 1000
    grid = (n // row_block,)
    ab = pl.pallas_call(
        _ab_body,
        grid=grid,
        in_specs=[
            pl.BlockSpec((row_block, d), lambda i: (i, 0)),
            pl.BlockSpec((d, d), lambda i: (0, 0)),
            pl.BlockSpec((d, d), lambda i: (0, 0)),
        ],
        out_specs=[
            pl.BlockSpec((row_block, d), lambda i: (i, 0)),
            pl.BlockSpec((row_block, d), lambda i: (i, 0)),
        ],
        out_shape=[
            jax.ShapeDtypeStruct((n, d), jnp.float32),
            jax.ShapeDtypeStruct((n, d), jnp.float32),
        ],
    )(feat, w1a, w1b)
    a, b = ab

    n_chunks = e // CHUNK
    src2d = edge_index[0].reshape(NW, n_chunks // NW, CHUNK)
    dst2d = edge_index[1].reshape(NW, n_chunks // NW, CHUNK)

    sc_edge = _make_sc_edge_kernel(n, d, n_chunks)
    partials = sc_edge(a, b, src2d, dst2d)

    bias2d = bias.reshape(1, d)
    out = pl.pallas_call(
        _final_body,
        grid=grid,
        in_specs=[
            pl.BlockSpec((2, row_block, d), lambda i: (0, i, 0)),
            pl.BlockSpec((row_block, d), lambda i: (i, 0)),
            pl.BlockSpec((d, d), lambda i: (0, 0)),
            pl.BlockSpec((d, d), lambda i: (0, 0)),
            pl.BlockSpec((1, d), lambda i: (0, 0)),
        ],
        out_specs=pl.BlockSpec((row_block, d), lambda i: (i, 0)),
        out_shape=jax.ShapeDtypeStruct((n, d), jnp.float32),
    )(partials, feat, weight2, loop_weight, bias2d)
    return out


# speedup vs baseline: 11.3246x; 1.5689x over previous
import functools

import jax
import jax.numpy as jnp
from jax import lax
from jax.experimental import pallas as pl
from jax.experimental.pallas import tpu as pltpu
from jax.experimental.pallas import tpu_sc as plsc

NC = 2
NS = 16
NW = NC * NS
LANES = 16

CHUNK = 80


def _ab_body(feat_ref, w1a_ref, w1b_ref, a_ref, b_ref):
    f = feat_ref[...]
    a_ref[...] = jnp.dot(f, w1a_ref[...], preferred_element_type=jnp.float32)
    b_ref[...] = jnp.dot(f, w1b_ref[...], preferred_element_type=jnp.float32)


def _final_body(s_ref, feat_ref, w2_ref, lw_ref, bias_ref, o_ref):
    s = s_ref[0] + s_ref[1]
    o_ref[...] = (
        jnp.dot(s, w2_ref[...], preferred_element_type=jnp.float32)
        + jnp.dot(feat_ref[...], lw_ref[...], preferred_element_type=jnp.float32)
        + bias_ref[...]
    )


def _make_sc_edge_kernel(n_nodes, d, n_chunks_total):
    chunks_per_worker = n_chunks_total // NW
    rows_per_tile = (n_nodes // NS) & ~7
    rows_rem = n_nodes - rows_per_tile * NS
    pieces = []
    off = 0
    while off < rows_per_tile:
        sz = min(CHUNK, rows_per_tile - off)
        pieces.append((off, sz))
        off += sz
    idx_rows = min(32, chunks_per_worker)
    phases = []
    off = 0
    while off < chunks_per_worker:
        sz = min(idx_rows, chunks_per_worker - off)
        phases.append((off, sz))
        off += sz
    mesh = plsc.VectorSubcoreMesh(core_axis_name="c", subcore_axis_name="s")

    @functools.partial(
        pl.kernel,
        out_type=jax.ShapeDtypeStruct((NC, n_nodes, d), jnp.float32),
        mesh=mesh,
        scratch_types=[
            pltpu.VMEM((idx_rows, CHUNK), jnp.int32),
            pltpu.VMEM((idx_rows, CHUNK), jnp.int32),
            pltpu.VMEM((CHUNK, d), jnp.float32),
            pltpu.VMEM((CHUNK, d), jnp.float32),
            pltpu.VMEM((CHUNK, d), jnp.float32),
            pltpu.VMEM_SHARED((n_nodes, d), jnp.float32),
            [pltpu.SemaphoreType.DMA] * 3,
            [pltpu.SemaphoreType.DMA] * 3,
            [pltpu.SemaphoreType.DMA] * 3,
        ],
    )
    def sc_edge(a_hbm, b_hbm, src_hbm, dst_hbm, out_hbm,
                idx_s, idx_d, rows0, rows1, rows2, acc, bsems, asems, ssems):
        cid = lax.axis_index("c")
        sid = lax.axis_index("s")
        wid = sid * NC + cid
        tile_base = sid * rows_per_tile

        zero = jnp.zeros((LANES,), jnp.float32)

        def zrow(r, carry):
            for cix in range(d // LANES):
                rows0[r, pl.ds(cix * LANES, LANES)] = zero
            return carry

        lax.fori_loop(0, CHUNK, zrow, 0)
        for poff, psz in pieces:
            pltpu.sync_copy(rows0.at[pl.ds(0, psz)],
                            acc.at[pl.ds(tile_base + poff, psz)])
        if rows_rem:
            @pl.when(sid == NS - 1)
            def _zero_rem():
                pltpu.sync_copy(rows0.at[pl.ds(0, rows_rem)],
                                acc.at[pl.ds(NS * rows_per_tile, rows_rem)])
        plsc.subcore_barrier()

        bufs = (rows0, rows1, rows2)

        def start_b(i, rr, sem):
            pltpu.async_copy(b_hbm.at[idx_d.at[i]], rr, sem)

        def wait_b(rr, sem):
            pltpu.make_async_copy(b_hbm.at[idx_d.at[0]], rr, sem).wait()

        def start_a(i, rr, sem):
            pltpu.async_copy(a_hbm.at[idx_s.at[i]], rr, sem, add=True)

        def wait_a(rr, sem):
            pltpu.make_async_copy(a_hbm.at[idx_s.at[0]], rr, sem).wait()

        def wait_scatter(rr, ssem):
            pltpu.make_async_copy(rr, acc.at[idx_d.at[0]], ssem).wait()

        def crunch(i, rr, ssem):
            def vrow(r, c2):
                for cix in range(d // LANES):
                    sl = pl.ds(cix * LANES, LANES)
                    rr[r, sl] = jnp.maximum(rr[r, sl], 0.0)
                return c2

            lax.fori_loop(0, CHUNK, vrow, 0)
            pltpu.async_copy(rr, acc.at[idx_d.at[i]], ssem, add=True)

        for phoff, phsz in phases:
            pltpu.sync_copy(src_hbm.at[wid].at[pl.ds(phoff, phsz)],
                            idx_s.at[pl.ds(0, phsz)])
            pltpu.sync_copy(dst_hbm.at[wid].at[pl.ds(phoff, phsz)],
                            idx_d.at[pl.ds(0, phsz)])
            start_b(0, bufs[0], bsems[0])
            if phsz > 1:
                start_b(1, bufs[1], bsems[1])
            wait_b(bufs[0], bsems[0])
            start_a(0, bufs[0], asems[0])

            def tstep(k, carry):
                for s in range(3):
                    j = 3 * k + s
                    x, y, z = s % 3, (s + 1) % 3, (s + 2) % 3

                    @pl.when(j < phsz)
                    def _do(j=j, x=x, y=y, z=z):
                        @pl.when(j >= 1)
                        def _wz():
                            wait_scatter(bufs[z], ssems[z])

                        @pl.when(j + 2 < phsz)
                        def _bz():
                            start_b(j + 2, bufs[z], bsems[z])

                        @pl.when(j + 1 < phsz)
                        def _ay():
                            wait_b(bufs[y], bsems[y])
                            start_a(j + 1, bufs[y], asems[y])

                        wait_a(bufs[x], asems[x])
                        crunch(j, bufs[x], ssems[x])

                return carry

            lax.fori_loop(0, (phsz + 2) // 3, tstep, 0)

            wait_scatter(bufs[(phsz - 1) % 3], ssems[(phsz - 1) % 3])

        plsc.subcore_barrier()
        for poff, psz in pieces:
            pltpu.sync_copy(acc.at[pl.ds(tile_base + poff, psz)],
                            rows0.at[pl.ds(0, psz)])
            pltpu.sync_copy(rows0.at[pl.ds(0, psz)],
                            out_hbm.at[cid].at[pl.ds(tile_base + poff, psz)])
        if rows_rem:
            @pl.when(sid == NS - 1)
            def _copy_rem():
                pltpu.sync_copy(acc.at[pl.ds(NS * rows_per_tile, rows_rem)],
                                rows0.at[pl.ds(0, rows_rem)])
                pltpu.sync_copy(rows0.at[pl.ds(0, rows_rem)],
                                out_hbm.at[cid].at[pl.ds(NS * rows_per_tile, rows_rem)])

    return sc_edge


def kernel(feat, edge_index, weight1, weight2, loop_weight, bias):
    n, d = feat.shape
    e = edge_index.shape[1]
    w1a = weight1[:d]
    w1b = weight1[d:]

    row_block = 1000
    grid = (n // row_block,)
    ab = pl.pallas_call(
        _ab_body,
        grid=grid,
        in_specs=[
            pl.BlockSpec((row_block, d), lambda i: (i, 0)),
            pl.BlockSpec((d, d), lambda i: (0, 0)),
            pl.BlockSpec((d, d), lambda i: (0, 0)),
        ],
        out_specs=[
            pl.BlockSpec((row_block, d), lambda i: (i, 0)),
            pl.BlockSpec((row_block, d), lambda i: (i, 0)),
        ],
        out_shape=[
            jax.ShapeDtypeStruct((n, d), jnp.float32),
            jax.ShapeDtypeStruct((n, d), jnp.float32),
        ],
    )(feat, w1a, w1b)
    a, b = ab

    n_chunks = e // CHUNK
    src2d = edge_index[0].reshape(NW, n_chunks // NW, CHUNK)
    dst2d = edge_index[1].reshape(NW, n_chunks // NW, CHUNK)

    sc_edge = _make_sc_edge_kernel(n, d, n_chunks)
    partials = sc_edge(a, b, src2d, dst2d)

    bias2d = bias.reshape(1, d)
    out = pl.pallas_call(
        _final_body,
        grid=grid,
        in_specs=[
            pl.BlockSpec((2, row_block, d), lambda i: (0, i, 0)),
            pl.BlockSpec((row_block, d), lambda i: (i, 0)),
            pl.BlockSpec((d, d), lambda i: (0, 0)),
            pl.BlockSpec((d, d), lambda i: (0, 0)),
            pl.BlockSpec((1, d), lambda i: (0, 0)),
        ],
        out_specs=pl.BlockSpec((row_block, d), lambda i: (i, 0)),
        out_shape=jax.ShapeDtypeStruct((n, d), jnp.float32),
    )(partials, feat, weight2, loop_weight, bias2d)
    return out
